# baseline (speedup 1.0000x reference)
import functools

import jax
import jax.numpy as jnp
from jax import lax
from jax.experimental import pallas as pl
from jax.experimental.pallas import tpu as pltpu
from jax.experimental.pallas import tpu_sc as plsc

_N = 10000
_E = 160000
_F = 128
_FE = 16
_FG = 16
_H = 256
_G = 64

_NP = 10240
_BN = 1000
_BE = 1000
_NC = 2
_NS = 16
_NW = _NC * _NS

_f32 = jnp.float32


def _pack2(lo, hi):
    lo16 = lax.bitcast_convert_type(lo.astype(jnp.bfloat16), jnp.uint16).astype(jnp.uint32)
    hi16 = lax.bitcast_convert_type(hi.astype(jnp.bfloat16), jnp.uint16).astype(jnp.uint32)
    return lax.bitcast_convert_type((hi16 << 16) | lo16, jnp.int32)


def _unpack2(v):
    u = lax.bitcast_convert_type(v, jnp.uint32)
    lo = lax.bitcast_convert_type(u << 16, _f32)
    hi = lax.bitcast_convert_type(u & jnp.uint32(0xFFFF0000), _f32)
    return lo, hi


def _node_pre_body(x_ref, oh_ref, u_ref, wcat_ref, wed_ref, eb1_ref,
                   n1b1_ref, t1_ref, xb_ref):
    x = x_ref[...]
    xc = jnp.dot(x, wcat_ref[...], preferred_element_type=_f32)
    ug2 = jnp.dot(u_ref[...], wed_ref[...], preferred_element_type=_f32) + eb1_ref[...]
    t1a = xc[:, :_H] + jnp.dot(oh_ref[...], ug2, preferred_element_type=_f32)
    t1b = xc[:, _H:2 * _H] + n1b1_ref[...]
    t1_ref[:, 0, :] = _pack2(t1a[:, :_F], t1a[:, _F:])
    t1_ref[:, 1, :] = _pack2(t1b[:, :_F], t1b[:, _F:])
    xb_ref[...] = _pack2(xc[:, 2 * _H:2 * _H + _F], xc[:, 2 * _H + _F:])


def _node_pre(x, oh, u, wcat, wed, eb1, n1b1):
    nb = _N // _BN
    return pl.pallas_call(
        _node_pre_body,
        grid=(nb,),
        in_specs=[
            pl.BlockSpec((_BN, _F), lambda i: (i, 0)),
            pl.BlockSpec((_BN, _G), lambda i: (i, 0)),
            pl.BlockSpec((_G, _FG), lambda i: (0, 0)),
            pl.BlockSpec((_F, 3 * _H), lambda i: (0, 0)),
            pl.BlockSpec((_FG, _H), lambda i: (0, 0)),
            pl.BlockSpec((1, _H), lambda i: (0, 0)),
            pl.BlockSpec((1, _H), lambda i: (0, 0)),
        ],
        out_specs=[
            pl.BlockSpec((_BN, 2, _F), lambda i: (i, 0, 0)),
            pl.BlockSpec((_BN, _F), lambda i: (i, 0)),
        ],
        out_shape=[
            jax.ShapeDtypeStruct((_N, 2, _F), jnp.int32),
            jax.ShapeDtypeStruct((_N, _F), jnp.int32),
        ],
    )(x, oh, u, wcat, wed, eb1, n1b1)


def _gather_body(t1_h, xb_h, row_h, col_h, o1_h, o2_h,
                 i1_v, i2_v, r1_v, r2_v, s1, s2):
    K = 40
    per_w = _E // _NW
    wid = lax.axis_index("s") * _NC + lax.axis_index("c")
    base0 = wid * per_w

    def body(i, carry):
        b = base0 + i * K
        pltpu.sync_copy(row_h.at[pl.ds(b, K)], i1_v)
        pltpu.sync_copy(col_h.at[pl.ds(b, K)], i2_v)
        c1 = pltpu.async_copy(t1_h.at[i1_v], r1_v, s1)
        c2 = pltpu.async_copy(xb_h.at[i2_v], r2_v, s2)
        c1.wait()
        c2.wait()
        pltpu.sync_copy(r1_v, o1_h.at[pl.ds(b, K)])
        pltpu.sync_copy(r2_v, o2_h.at[pl.ds(b, K)])
        return carry

    lax.fori_loop(0, per_w // K, body, 0)


def _sc_gather(t1, xb, row, col):
    K = 40
    f = pl.kernel(
        _gather_body,
        out_type=[
            jax.ShapeDtypeStruct((_E, 2, _F), jnp.int32),
            jax.ShapeDtypeStruct((_E, _F), jnp.int32),
        ],
        mesh=plsc.VectorSubcoreMesh(core_axis_name="c", subcore_axis_name="s"),
        scratch_types=[
            pltpu.VMEM((K,), jnp.int32),
            pltpu.VMEM((K,), jnp.int32),
            pltpu.VMEM((K, 2, _F), jnp.int32),
            pltpu.VMEM((K, _F), jnp.int32),
            pltpu.SemaphoreType.DMA,
            pltpu.SemaphoreType.DMA,
        ],
    )
    return f(t1, xb, row, col)


def _edge_body(g1_ref, g2_ref, ea_ref, wec_ref, ew2_ref, eb2_ref, wn1b_ref,
               ean_ref, h1n_ref):
    eaw = jnp.dot(ea_ref[...], wec_ref[...], preferred_element_type=_f32)
    t1a_lo, t1a_hi = _unpack2(g1_ref[:, 0, :])
    t1b_lo, t1b_hi = _unpack2(g1_ref[:, 1, :])
    xb_lo, xb_hi = _unpack2(g2_ref[...])
    h1e_lo = jnp.maximum(t1a_lo + xb_lo + eaw[:, :_F], 0.0)
    h1e_hi = jnp.maximum(t1a_hi + xb_hi + eaw[:, _F:], 0.0)
    ean = (jnp.dot(h1e_lo, ew2_ref[:_F], preferred_element_type=_f32)
           + jnp.dot(h1e_hi, ew2_ref[_F:], preferred_element_type=_f32)
           + eb2_ref[...])
    eanw = jnp.dot(ean, wn1b_ref[...], preferred_element_type=_f32)
    ean_ref[...] = ean
    h1n_ref[0] = jnp.maximum(t1b_lo + eanw[:, :_F], 0.0)
    h1n_ref[1] = jnp.maximum(t1b_hi + eanw[:, _F:], 0.0)


def _edge_dense(g1, g2, ea, wec, ew2, eb2, wn1b):
    nb = _E // _BE
    return pl.pallas_call(
        _edge_body,
        grid=(nb,),
        in_specs=[
            pl.BlockSpec((_BE, 2, _F), lambda i: (i, 0, 0)),
            pl.BlockSpec((_BE, _F), lambda i: (i, 0)),
            pl.BlockSpec((_BE, _FE), lambda i: (i, 0)),
            pl.BlockSpec((_FE, _H), lambda i: (0, 0)),
            pl.BlockSpec((_H, _FE), lambda i: (0, 0)),
            pl.BlockSpec((1, _FE), lambda i: (0, 0)),
            pl.BlockSpec((_FE, _H), lambda i: (0, 0)),
        ],
        out_specs=[
            pl.BlockSpec((_BE, _FE), lambda i: (i, 0)),
            pl.BlockSpec((2, _BE, _F), lambda i: (0, i, 0)),
        ],
        out_shape=[
            jax.ShapeDtypeStruct((_E, _FE), _f32),
            jax.ShapeDtypeStruct((2, _E, _F), _f32),
        ],
    )(g1, g2, ea, wec, ew2, eb2, wn1b)


def _scatter_body(h1n_h, col_h, zer_h, out_h, idx_v, buf_v, acc_sh):
    K = 80
    rows_per_tile = _NP // _NS
    per_tile = _E // _NS
    c = lax.axis_index("c")
    s = lax.axis_index("s")

    pltpu.sync_copy(zer_h.at[pl.ds(s * rows_per_tile, rows_per_tile)],
                    acc_sh.at[pl.ds(s * rows_per_tile, rows_per_tile)])
    plsc.subcore_barrier()

    def body(i, carry):
        b = s * per_tile + i * K
        pltpu.sync_copy(col_h.at[pl.ds(b, K)], idx_v)
        pltpu.sync_copy(h1n_h.at[c, pl.ds(b, K)], buf_v)
        pltpu.sync_copy(buf_v, acc_sh.at[idx_v], add=True)
        return carry

    lax.fori_loop(0, per_tile // K, body, 0)
    plsc.subcore_barrier()
    pltpu.sync_copy(acc_sh.at[pl.ds(s * rows_per_tile, rows_per_tile)],
                    out_h.at[c, pl.ds(s * rows_per_tile, rows_per_tile)])


def _sc_scatter(h1n, col, zer):
    K = 80
    f = pl.kernel(
        _scatter_body,
        out_type=jax.ShapeDtypeStruct((2, _NP, _F), _f32),
        mesh=plsc.VectorSubcoreMesh(core_axis_name="c", subcore_axis_name="s"),
        scratch_types=[
            pltpu.VMEM((K,), jnp.int32),
            pltpu.VMEM((K, _F), _f32),
            pltpu.VMEM_SHARED((_NP, _F), _f32),
        ],
    )
    return f(h1n, col, zer)


def _deg_body(col_h, zer_h, ones_h, out_h, idx_v, ones_v, acc_sh):
    K = 80
    rows_per_tile = _NP // _NS
    per_tile = _E // _NS
    c = lax.axis_index("c")
    s = lax.axis_index("s")

    @pl.when(c == 0)
    def _():
        pltpu.sync_copy(ones_h, ones_v)
        pltpu.sync_copy(zer_h.at[pl.ds(s * rows_per_tile, rows_per_tile)],
                        acc_sh.at[pl.ds(s * rows_per_tile, rows_per_tile)])
        plsc.subcore_barrier()

        def body(i, carry):
            b = s * per_tile + i * K
            pltpu.sync_copy(col_h.at[pl.ds(b, K)], idx_v)
            pltpu.sync_copy(ones_v, acc_sh.at[idx_v], add=True)
            return carry

        lax.fori_loop(0, per_tile // K, body, 0)
        plsc.subcore_barrier()
        pltpu.sync_copy(acc_sh.at[pl.ds(s * rows_per_tile, rows_per_tile)],
                        out_h.at[pl.ds(s * rows_per_tile, rows_per_tile)])


def _sc_degree(col, zer):
    K = 80
    ones = jnp.ones((K, _F), _f32)
    f = pl.kernel(
        _deg_body,
        out_type=jax.ShapeDtypeStruct((_NP, _F), _f32),
        mesh=plsc.VectorSubcoreMesh(core_axis_name="c", subcore_axis_name="s"),
        scratch_types=[
            pltpu.VMEM((K,), jnp.int32),
            pltpu.VMEM((K, _F), _f32),
            pltpu.VMEM_SHARED((_NP, _F), _f32),
        ],
    )
    return f(col, zer, ones)


def _node_post_body(x_ref, agg_ref, deg_ref, oh_ref, u_ref,
                    n1w2_ref, n1b2_ref, wn2a_ref, wn2b_ref, wn2c_ref,
                    n2b1_ref, n2w2_ref, n2b2_ref, xn_ref):
    deg = deg_ref[:, :1]
    aggf = jnp.concatenate([agg_ref[0], agg_ref[1]], axis=1)
    aggm = aggf / jnp.maximum(deg, 1.0)
    aggv = (jnp.dot(aggm, n1w2_ref[...], preferred_element_type=_f32)
            + n1b2_ref[...] * jnp.minimum(deg, 1.0))
    ub = jnp.dot(oh_ref[...],
                 jnp.dot(u_ref[...], wn2c_ref[...], preferred_element_type=_f32),
                 preferred_element_type=_f32)
    h2 = jnp.maximum(
        jnp.dot(x_ref[...], wn2a_ref[...], preferred_element_type=_f32)
        + jnp.dot(aggv, wn2b_ref[...], preferred_element_type=_f32)
        + ub + n2b1_ref[...], 0.0)
    xn_ref[...] = jnp.dot(h2, n2w2_ref[...], preferred_element_type=_f32) + n2b2_ref[...]


def _node_post(x, agg, deg16, oh, u, n1w2, n1b2, wn2a, wn2b, wn2c, n2b1, n2w2, n2b2):
    nb = _N // _BN
    return pl.pallas_call(
        _node_post_body,
        grid=(nb,),
        in_specs=[
            pl.BlockSpec((_BN, _F), lambda i: (i, 0)),
            pl.BlockSpec((2, _BN, _F), lambda i: (0, i, 0)),
            pl.BlockSpec((_BN, _F), lambda i: (i, 0)),
            pl.BlockSpec((_BN, _G), lambda i: (i, 0)),
            pl.BlockSpec((_G, _FG), lambda i: (0, 0)),
            pl.BlockSpec((_H, _H), lambda i: (0, 0)),
            pl.BlockSpec((1, _H), lambda i: (0, 0)),
            pl.BlockSpec((_F, _H), lambda i: (0, 0)),
            pl.BlockSpec((_H, _H), lambda i: (0, 0)),
            pl.BlockSpec((_FG, _H), lambda i: (0, 0)),
            pl.BlockSpec((1, _H), lambda i: (0, 0)),
            pl.BlockSpec((_H, _F), lambda i: (0, 0)),
            pl.BlockSpec((1, _F), lambda i: (0, 0)),
        ],
        out_specs=pl.BlockSpec((_BN, _F), lambda i: (i, 0)),
        out_shape=jax.ShapeDtypeStruct((_N, _F), _f32),
    )(x, agg, deg16, oh, u, n1w2, n1b2, wn2a, wn2b, wn2c, n2b1, n2w2, n2b2)


def _glob_body(xn_ref, oh_ref, u_ref, wga_ref, wgb_ref, gb1_ref, gw2_ref,
               gb2_ref, un_ref):
    oh = oh_ref[...]
    xs = lax.dot_general(oh, xn_ref[...], (((0,), (0,)), ((), ())),
                         preferred_element_type=_f32)
    cnt = jnp.sum(oh, axis=0)[:, None]
    xm = xs / jnp.maximum(cnt, 1.0)
    h = jnp.maximum(
        jnp.dot(u_ref[...], wga_ref[...], preferred_element_type=_f32)
        + jnp.dot(xm, wgb_ref[...], preferred_element_type=_f32)
        + gb1_ref[...], 0.0)
    un_ref[...] = jnp.dot(h, gw2_ref[...], preferred_element_type=_f32) + gb2_ref[...]


def _glob_update(xn, oh, u, wga, wgb, gb1, gw2, gb2):
    return pl.pallas_call(
        _glob_body,
        out_shape=jax.ShapeDtypeStruct((_G, _FG), _f32),
    )(xn, oh, u, wga, wgb, gb1, gw2, gb2)


def kernel(x, edge_index, edge_attr, u, batch,
           edge_w1, edge_b1, edge_w2, edge_b2,
           node1_w1, node1_b1, node1_w2, node1_b2,
           node2_w1, node2_b1, node2_w2, node2_b2,
           glob_w1, glob_b1, glob_w2, glob_b2):
    row = edge_index[0].astype(jnp.int32)
    col = edge_index[1].astype(jnp.int32)

    oh = (batch[:, None] == jnp.arange(_G, dtype=batch.dtype)[None, :]).astype(_f32)

    wea = edge_w1[:_F]
    web = edge_w1[_F:2 * _F]
    wec = edge_w1[2 * _F:2 * _F + _FE]
    wed = edge_w1[2 * _F + _FE:]
    wn1a = node1_w1[:_F]
    wn1b = node1_w1[_F:]
    wn2a = node2_w1[:_F]
    wn2b = node2_w1[_F:_F + _H]
    wn2c = node2_w1[_F + _H:]
    wga = glob_w1[:_FG]
    wgb = glob_w1[_FG:]
    wcat = jnp.concatenate([wea, wn1a, web], axis=1)

    eb1 = edge_b1[None, :]
    n1b1 = node1_b1[None, :]
    eb2 = edge_b2[None, :]
    n1b2 = node1_b2[None, :]
    n2b1 = node2_b1[None, :]
    n2b2 = node2_b2[None, :]
    gb1 = glob_b1[None, :]
    gb2 = glob_b2[None, :]

    zer = jnp.zeros((_NP, _F), _f32)

    deg16 = _sc_degree(col, zer)

    for _ in range(3):
        t1, xb = _node_pre(x, oh, u, wcat, wed, eb1, n1b1)
        g1, g2 = _sc_gather(t1, xb, row, col)
        ean, h1n = _edge_dense(g1, g2, edge_attr, wec, edge_w2, eb2, wn1b)
        agg = _sc_scatter(h1n, col, zer)
        x_new = _node_post(x, agg, deg16, oh, u, node1_w2, n1b2,
                           wn2a, wn2b, wn2c, n2b1, node2_w2, n2b2)
        u = _glob_update(x_new, oh, u, wga, wgb, gb1, gw2=glob_w2, gb2=gb2)
        x = x_new
        edge_attr = ean

    return (x, edge_attr, u)


# speedup vs baseline: 1.3375x; 1.3375x over previous
import functools

import jax
import jax.numpy as jnp
from jax import lax
from jax.experimental import pallas as pl
from jax.experimental.pallas import tpu as pltpu
from jax.experimental.pallas import tpu_sc as plsc

_N = 10000
_E = 160000
_F = 128
_FE = 16
_FG = 16
_H = 256
_G = 64

_NP = 10240
_BN = 1000
_BE = 1000
_NC = 2
_NS = 16
_NW = _NC * _NS

_f32 = jnp.float32


def _pack2(lo, hi):
    lo16 = lax.bitcast_convert_type(lo.astype(jnp.bfloat16), jnp.uint16).astype(jnp.uint32)
    hi16 = lax.bitcast_convert_type(hi.astype(jnp.bfloat16), jnp.uint16).astype(jnp.uint32)
    return lax.bitcast_convert_type((hi16 << 16) | lo16, jnp.int32)


def _unpack2(v):
    u = lax.bitcast_convert_type(v, jnp.uint32)
    lo = lax.bitcast_convert_type(u << 16, _f32)
    hi = lax.bitcast_convert_type(u & jnp.uint32(0xFFFF0000), _f32)
    return lo, hi


def _node_pre_body(x_ref, oh_ref, u_ref, wcat_ref, wed_ref, eb1_ref,
                   n1b1_ref, t1_ref, xb_ref):
    x = x_ref[...]
    xc = jnp.dot(x, wcat_ref[...], preferred_element_type=_f32)
    ug2 = jnp.dot(u_ref[...], wed_ref[...], preferred_element_type=_f32) + eb1_ref[...]
    t1a = xc[:, :_H] + jnp.dot(oh_ref[...], ug2, preferred_element_type=_f32)
    t1b = xc[:, _H:2 * _H] + n1b1_ref[...]
    t1_ref[:, 0, :] = _pack2(t1a[:, :_F], t1a[:, _F:])
    t1_ref[:, 1, :] = _pack2(t1b[:, :_F], t1b[:, _F:])
    xb_ref[...] = _pack2(xc[:, 2 * _H:2 * _H + _F], xc[:, 2 * _H + _F:])


def _node_pre(x, oh, u, wcat, wed, eb1, n1b1):
    nb = _N // _BN
    return pl.pallas_call(
        _node_pre_body,
        grid=(nb,),
        in_specs=[
            pl.BlockSpec((_BN, _F), lambda i: (i, 0)),
            pl.BlockSpec((_BN, _G), lambda i: (i, 0)),
            pl.BlockSpec((_G, _FG), lambda i: (0, 0)),
            pl.BlockSpec((_F, 3 * _H), lambda i: (0, 0)),
            pl.BlockSpec((_FG, _H), lambda i: (0, 0)),
            pl.BlockSpec((1, _H), lambda i: (0, 0)),
            pl.BlockSpec((1, _H), lambda i: (0, 0)),
        ],
        out_specs=[
            pl.BlockSpec((_BN, 2, _F), lambda i: (i, 0, 0)),
            pl.BlockSpec((_BN, _F), lambda i: (i, 0)),
        ],
        out_shape=[
            jax.ShapeDtypeStruct((_N, 2, _F), jnp.int32),
            jax.ShapeDtypeStruct((_N, _F), jnp.int32),
        ],
    )(x, oh, u, wcat, wed, eb1, n1b1)


_GK = 40
_GIT = (_E // _NW) // _GK


def _gather_body(t1_h, xb_h, row_h, col_h, o1_h, o2_h,
                 i1_v, i2_v, r1a, r2a, r1b, r2b, sa1, sa2, sb1, sb2):
    K = _GK
    per_w = _E // _NW
    wid = lax.axis_index("s") * _NC + lax.axis_index("c")
    base0 = wid * per_w

    pltpu.sync_copy(row_h.at[wid], i1_v)
    pltpu.sync_copy(col_h.at[wid], i2_v)

    def fire(j, r1, r2, s1, s2):
        pltpu.async_copy(t1_h.at[i1_v.at[j]], r1, s1)
        pltpu.async_copy(xb_h.at[i2_v.at[j]], r2, s2)

    def wait(r1, r2, s1, s2):
        pltpu.make_async_copy(t1_h.at[i1_v.at[0]], r1, s1).wait()
        pltpu.make_async_copy(xb_h.at[i2_v.at[0]], r2, s2).wait()

    def store(j, r1, r2):
        b = base0 + j * K
        pltpu.sync_copy(r1, o1_h.at[pl.ds(b, K)])
        pltpu.sync_copy(r2, o2_h.at[pl.ds(b, K)])

    fire(0, r1a, r2a, sa1, sa2)

    def body(g, carry):
        j = 2 * g
        fire(j + 1, r1b, r2b, sb1, sb2)
        wait(r1a, r2a, sa1, sa2)
        store(j, r1a, r2a)
        fire(j + 2, r1a, r2a, sa1, sa2)
        wait(r1b, r2b, sb1, sb2)
        store(j + 1, r1b, r2b)
        return carry

    lax.fori_loop(0, (_GIT - 1) // 2, body, 0)
    wait(r1a, r2a, sa1, sa2)
    store(_GIT - 1, r1a, r2a)


def _sc_gather(t1, xb, row_r, col_r):
    K = _GK
    f = pl.kernel(
        _gather_body,
        out_type=[
            jax.ShapeDtypeStruct((_E, 2, _F), jnp.int32),
            jax.ShapeDtypeStruct((_E, _F), jnp.int32),
        ],
        mesh=plsc.VectorSubcoreMesh(core_axis_name="c", subcore_axis_name="s"),
        scratch_types=[
            pltpu.VMEM((_GIT, K), jnp.int32),
            pltpu.VMEM((_GIT, K), jnp.int32),
            pltpu.VMEM((K, 2, _F), jnp.int32),
            pltpu.VMEM((K, _F), jnp.int32),
            pltpu.VMEM((K, 2, _F), jnp.int32),
            pltpu.VMEM((K, _F), jnp.int32),
            pltpu.SemaphoreType.DMA,
            pltpu.SemaphoreType.DMA,
            pltpu.SemaphoreType.DMA,
            pltpu.SemaphoreType.DMA,
        ],
    )
    return f(t1, xb, row_r, col_r)


def _edge_body(g1_ref, g2_ref, ea_ref, wec_ref, ew2_ref, eb2_ref, wn1b_ref,
               ean_ref, h1n_ref):
    eaw = jnp.dot(ea_ref[...], wec_ref[...], preferred_element_type=_f32)
    t1a_lo, t1a_hi = _unpack2(g1_ref[:, 0, :])
    t1b_lo, t1b_hi = _unpack2(g1_ref[:, 1, :])
    xb_lo, xb_hi = _unpack2(g2_ref[...])
    h1e_lo = jnp.maximum(t1a_lo + xb_lo + eaw[:, :_F], 0.0)
    h1e_hi = jnp.maximum(t1a_hi + xb_hi + eaw[:, _F:], 0.0)
    ean = (jnp.dot(h1e_lo, ew2_ref[:_F], preferred_element_type=_f32)
           + jnp.dot(h1e_hi, ew2_ref[_F:], preferred_element_type=_f32)
           + eb2_ref[...])
    eanw = jnp.dot(ean, wn1b_ref[...], preferred_element_type=_f32)
    ean_ref[...] = ean
    h1n_ref[0] = jnp.maximum(t1b_lo + eanw[:, :_F], 0.0)
    h1n_ref[1] = jnp.maximum(t1b_hi + eanw[:, _F:], 0.0)


def _edge_dense(g1, g2, ea, wec, ew2, eb2, wn1b):
    nb = _E // _BE
    return pl.pallas_call(
        _edge_body,
        grid=(nb,),
        in_specs=[
            pl.BlockSpec((_BE, 2, _F), lambda i: (i, 0, 0)),
            pl.BlockSpec((_BE, _F), lambda i: (i, 0)),
            pl.BlockSpec((_BE, _FE), lambda i: (i, 0)),
            pl.BlockSpec((_FE, _H), lambda i: (0, 0)),
            pl.BlockSpec((_H, _FE), lambda i: (0, 0)),
            pl.BlockSpec((1, _FE), lambda i: (0, 0)),
            pl.BlockSpec((_FE, _H), lambda i: (0, 0)),
        ],
        out_specs=[
            pl.BlockSpec((_BE, _FE), lambda i: (i, 0)),
            pl.BlockSpec((2, _BE, _F), lambda i: (0, i, 0)),
        ],
        out_shape=[
            jax.ShapeDtypeStruct((_E, _FE), _f32),
            jax.ShapeDtypeStruct((2, _E, _F), _f32),
        ],
    )(g1, g2, ea, wec, ew2, eb2, wn1b)


_SK = 80
_SIT = (_E // _NS) // _SK


def _scatter_body(h1n_h, col_h, zer_h, out_h, idx_v, bufa, bufb, acc_sh, sa, sb):
    K = _SK
    rows_per_tile = _NP // _NS
    per_tile = _E // _NS
    c = lax.axis_index("c")
    s = lax.axis_index("s")

    pltpu.sync_copy(zer_h.at[pl.ds(s * rows_per_tile, rows_per_tile)],
                    acc_sh.at[pl.ds(s * rows_per_tile, rows_per_tile)])
    pltpu.sync_copy(col_h.at[s], idx_v)
    plsc.subcore_barrier()

    def fire(j, buf, sem):
        pltpu.async_copy(h1n_h.at[c, pl.ds(s * per_tile + j * K, K)], buf, sem)

    def wait(buf, sem):
        pltpu.make_async_copy(h1n_h.at[c, pl.ds(0, K)], buf, sem).wait()

    def scat(j, buf):
        pltpu.sync_copy(buf, acc_sh.at[idx_v.at[j]], add=True)

    fire(0, bufa, sa)

    def body(g, carry):
        j = 2 * g
        fire(j + 1, bufb, sb)
        wait(bufa, sa)
        scat(j, bufa)
        fire(j + 2, bufa, sa)
        wait(bufb, sb)
        scat(j + 1, bufb)
        return carry

    lax.fori_loop(0, (_SIT - 1) // 2, body, 0)
    wait(bufa, sa)
    scat(_SIT - 1, bufa)

    plsc.subcore_barrier()
    pltpu.sync_copy(acc_sh.at[pl.ds(s * rows_per_tile, rows_per_tile)],
                    out_h.at[c, pl.ds(s * rows_per_tile, rows_per_tile)])


def _sc_scatter(h1n, col_r, zer):
    K = _SK
    f = pl.kernel(
        _scatter_body,
        out_type=jax.ShapeDtypeStruct((2, _NP, _F), _f32),
        mesh=plsc.VectorSubcoreMesh(core_axis_name="c", subcore_axis_name="s"),
        scratch_types=[
            pltpu.VMEM((_SIT, K), jnp.int32),
            pltpu.VMEM((K, _F), _f32),
            pltpu.VMEM((K, _F), _f32),
            pltpu.VMEM_SHARED((_NP, _F), _f32),
            pltpu.SemaphoreType.DMA,
            pltpu.SemaphoreType.DMA,
        ],
    )
    return f(h1n, col_r, zer)


def _deg_body(col_h, zer_h, ones_h, out_h, idx_v, ones_v, acc_sh):
    K = 80
    rows_per_tile = _NP // _NS
    per_tile = _E // _NS
    c = lax.axis_index("c")
    s = lax.axis_index("s")

    @pl.when(c == 0)
    def _():
        pltpu.sync_copy(ones_h, ones_v)
        pltpu.sync_copy(zer_h.at[pl.ds(s * rows_per_tile, rows_per_tile)],
                        acc_sh.at[pl.ds(s * rows_per_tile, rows_per_tile)])
        pltpu.sync_copy(col_h.at[s], idx_v)
        plsc.subcore_barrier()

        def body(i, carry):
            pltpu.sync_copy(ones_v, acc_sh.at[idx_v.at[i]], add=True)
            return carry

        lax.fori_loop(0, per_tile // K, body, 0)
        plsc.subcore_barrier()
        pltpu.sync_copy(acc_sh.at[pl.ds(s * rows_per_tile, rows_per_tile)],
                        out_h.at[pl.ds(s * rows_per_tile, rows_per_tile)])


def _sc_degree(col_r, zer):
    K = 80
    ones = jnp.ones((K, _F), _f32)
    f = pl.kernel(
        _deg_body,
        out_type=jax.ShapeDtypeStruct((_NP, _F), _f32),
        mesh=plsc.VectorSubcoreMesh(core_axis_name="c", subcore_axis_name="s"),
        scratch_types=[
            pltpu.VMEM((_SIT, K), jnp.int32),
            pltpu.VMEM((K, _F), _f32),
            pltpu.VMEM_SHARED((_NP, _F), _f32),
        ],
    )
    return f(col_r, zer, ones)


def _node_post_body(x_ref, agg_ref, deg_ref, oh_ref, u_ref,
                    n1w2_ref, n1b2_ref, wn2a_ref, wn2b_ref, wn2c_ref,
                    n2b1_ref, n2w2_ref, n2b2_ref, xn_ref):
    deg = deg_ref[:, :1]
    aggf = jnp.concatenate([agg_ref[0], agg_ref[1]], axis=1)
    aggm = aggf / jnp.maximum(deg, 1.0)
    aggv = (jnp.dot(aggm, n1w2_ref[...], preferred_element_type=_f32)
            + n1b2_ref[...] * jnp.minimum(deg, 1.0))
    ub = jnp.dot(oh_ref[...],
                 jnp.dot(u_ref[...], wn2c_ref[...], preferred_element_type=_f32),
                 preferred_element_type=_f32)
    h2 = jnp.maximum(
        jnp.dot(x_ref[...], wn2a_ref[...], preferred_element_type=_f32)
        + jnp.dot(aggv, wn2b_ref[...], preferred_element_type=_f32)
        + ub + n2b1_ref[...], 0.0)
    xn_ref[...] = jnp.dot(h2, n2w2_ref[...], preferred_element_type=_f32) + n2b2_ref[...]


def _node_post(x, agg, deg16, oh, u, n1w2, n1b2, wn2a, wn2b, wn2c, n2b1, n2w2, n2b2):
    nb = _N // _BN
    return pl.pallas_call(
        _node_post_body,
        grid=(nb,),
        in_specs=[
            pl.BlockSpec((_BN, _F), lambda i: (i, 0)),
            pl.BlockSpec((2, _BN, _F), lambda i: (0, i, 0)),
            pl.BlockSpec((_BN, _F), lambda i: (i, 0)),
            pl.BlockSpec((_BN, _G), lambda i: (i, 0)),
            pl.BlockSpec((_G, _FG), lambda i: (0, 0)),
            pl.BlockSpec((_H, _H), lambda i: (0, 0)),
            pl.BlockSpec((1, _H), lambda i: (0, 0)),
            pl.BlockSpec((_F, _H), lambda i: (0, 0)),
            pl.BlockSpec((_H, _H), lambda i: (0, 0)),
            pl.BlockSpec((_FG, _H), lambda i: (0, 0)),
            pl.BlockSpec((1, _H), lambda i: (0, 0)),
            pl.BlockSpec((_H, _F), lambda i: (0, 0)),
            pl.BlockSpec((1, _F), lambda i: (0, 0)),
        ],
        out_specs=pl.BlockSpec((_BN, _F), lambda i: (i, 0)),
        out_shape=jax.ShapeDtypeStruct((_N, _F), _f32),
    )(x, agg, deg16, oh, u, n1w2, n1b2, wn2a, wn2b, wn2c, n2b1, n2w2, n2b2)


def _glob_body(xn_ref, oh_ref, u_ref, wga_ref, wgb_ref, gb1_ref, gw2_ref,
               gb2_ref, un_ref):
    oh = oh_ref[...]
    xs = lax.dot_general(oh, xn_ref[...], (((0,), (0,)), ((), ())),
                         preferred_element_type=_f32)
    cnt = jnp.sum(oh, axis=0)[:, None]
    xm = xs / jnp.maximum(cnt, 1.0)
    h = jnp.maximum(
        jnp.dot(u_ref[...], wga_ref[...], preferred_element_type=_f32)
        + jnp.dot(xm, wgb_ref[...], preferred_element_type=_f32)
        + gb1_ref[...], 0.0)
    un_ref[...] = jnp.dot(h, gw2_ref[...], preferred_element_type=_f32) + gb2_ref[...]


def _glob_update(xn, oh, u, wga, wgb, gb1, gw2, gb2):
    return pl.pallas_call(
        _glob_body,
        out_shape=jax.ShapeDtypeStruct((_G, _FG), _f32),
    )(xn, oh, u, wga, wgb, gb1, gw2, gb2)


def kernel(x, edge_index, edge_attr, u, batch,
           edge_w1, edge_b1, edge_w2, edge_b2,
           node1_w1, node1_b1, node1_w2, node1_b2,
           node2_w1, node2_b1, node2_w2, node2_b2,
           glob_w1, glob_b1, glob_w2, glob_b2):
    row = edge_index[0].astype(jnp.int32)
    col = edge_index[1].astype(jnp.int32)
    row_g = row.reshape(_NW, _GIT, _GK)
    col_g = col.reshape(_NW, _GIT, _GK)
    col_s = col.reshape(_NS, _SIT, _SK)

    oh = (batch[:, None] == jnp.arange(_G, dtype=batch.dtype)[None, :]).astype(_f32)

    wea = edge_w1[:_F]
    web = edge_w1[_F:2 * _F]
    wec = edge_w1[2 * _F:2 * _F + _FE]
    wed = edge_w1[2 * _F + _FE:]
    wn1a = node1_w1[:_F]
    wn1b = node1_w1[_F:]
    wn2a = node2_w1[:_F]
    wn2b = node2_w1[_F:_F + _H]
    wn2c = node2_w1[_F + _H:]
    wga = glob_w1[:_FG]
    wgb = glob_w1[_FG:]
    wcat = jnp.concatenate([wea, wn1a, web], axis=1)

    eb1 = edge_b1[None, :]
    n1b1 = node1_b1[None, :]
    eb2 = edge_b2[None, :]
    n1b2 = node1_b2[None, :]
    n2b1 = node2_b1[None, :]
    n2b2 = node2_b2[None, :]
    gb1 = glob_b1[None, :]
    gb2 = glob_b2[None, :]

    zer = jnp.zeros((_NP, _F), _f32)

    deg16 = _sc_degree(col_s, zer)

    for _ in range(3):
        t1, xb = _node_pre(x, oh, u, wcat, wed, eb1, n1b1)
        g1, g2 = _sc_gather(t1, xb, row_g, col_g)
        ean, h1n = _edge_dense(g1, g2, edge_attr, wec, edge_w2, eb2, wn1b)
        agg = _sc_scatter(h1n, col_s, zer)
        x_new = _node_post(x, agg, deg16, oh, u, node1_w2, n1b2,
                           wn2a, wn2b, wn2c, n2b1, node2_w2, n2b2)
        u = _glob_update(x_new, oh, u, wga, wgb, gb1, gw2=glob_w2, gb2=gb2)
        x = x_new
        edge_attr = ean

    return (x, edge_attr, u)


# speedup vs baseline: 1.3606x; 1.0172x over previous
import functools

import jax
import jax.numpy as jnp
from jax import lax
from jax.experimental import pallas as pl
from jax.experimental.pallas import tpu as pltpu
from jax.experimental.pallas import tpu_sc as plsc

_N = 10000
_E = 160000
_F = 128
_FE = 16
_FG = 16
_H = 256
_G = 64

_NP = 10240
_BN = 2000
_BE = 2000
_NC = 2
_NS = 16
_NW = _NC * _NS

_f32 = jnp.float32


def _pack2(lo, hi):
    lo16 = lax.bitcast_convert_type(lo.astype(jnp.bfloat16), jnp.uint16).astype(jnp.uint32)
    hi16 = lax.bitcast_convert_type(hi.astype(jnp.bfloat16), jnp.uint16).astype(jnp.uint32)
    return lax.bitcast_convert_type((hi16 << 16) | lo16, jnp.int32)


def _unpack2(v):
    u = lax.bitcast_convert_type(v, jnp.uint32)
    lo = lax.bitcast_convert_type(u << 16, _f32)
    hi = lax.bitcast_convert_type(u & jnp.uint32(0xFFFF0000), _f32)
    return lo, hi


def _node_pre_body(x_ref, oh_ref, u_ref, wcat_ref, wed_ref, eb1_ref,
                   n1b1_ref, t1_ref, xb_ref):
    x = x_ref[...]
    xc = jnp.dot(x, wcat_ref[...], preferred_element_type=_f32)
    ug2 = jnp.dot(u_ref[...], wed_ref[...], preferred_element_type=_f32) + eb1_ref[...]
    t1a = xc[:, :_H] + jnp.dot(oh_ref[...], ug2, preferred_element_type=_f32)
    t1b = xc[:, _H:2 * _H] + n1b1_ref[...]
    t1_ref[:, 0, :] = _pack2(t1a[:, :_F], t1a[:, _F:])
    t1_ref[:, 1, :] = _pack2(t1b[:, :_F], t1b[:, _F:])
    xb_ref[...] = _pack2(xc[:, 2 * _H:2 * _H + _F], xc[:, 2 * _H + _F:])


def _node_pre(x, oh, u, wcat, wed, eb1, n1b1):
    nb = _N // _BN
    return pl.pallas_call(
        _node_pre_body,
        grid=(nb,),
        in_specs=[
            pl.BlockSpec((_BN, _F), lambda i: (i, 0)),
            pl.BlockSpec((_BN, _G), lambda i: (i, 0)),
            pl.BlockSpec((_G, _FG), lambda i: (0, 0)),
            pl.BlockSpec((_F, 3 * _H), lambda i: (0, 0)),
            pl.BlockSpec((_FG, _H), lambda i: (0, 0)),
            pl.BlockSpec((1, _H), lambda i: (0, 0)),
            pl.BlockSpec((1, _H), lambda i: (0, 0)),
        ],
        out_specs=[
            pl.BlockSpec((_BN, 2, _F), lambda i: (i, 0, 0)),
            pl.BlockSpec((_BN, _F), lambda i: (i, 0)),
        ],
        out_shape=[
            jax.ShapeDtypeStruct((_N, 2, _F), jnp.int32),
            jax.ShapeDtypeStruct((_N, _F), jnp.int32),
        ],
    )(x, oh, u, wcat, wed, eb1, n1b1)


_GK = 40
_GIT = (_E // _NW) // _GK


def _gather_body(t1_h, xb_h, row_h, col_h, o1_h, o2_h,
                 i1_v, i2_v, r1a, r2a, r1b, r2b, sa1, sa2, sb1, sb2):
    K = _GK
    per_w = _E // _NW
    wid = lax.axis_index("s") * _NC + lax.axis_index("c")
    base0 = wid * per_w

    pltpu.sync_copy(row_h.at[wid], i1_v)
    pltpu.sync_copy(col_h.at[wid], i2_v)

    def fire(j, r1, r2, s1, s2):
        pltpu.async_copy(t1_h.at[i1_v.at[j]], r1, s1)
        pltpu.async_copy(xb_h.at[i2_v.at[j]], r2, s2)

    def wait(r1, r2, s1, s2):
        pltpu.make_async_copy(t1_h.at[i1_v.at[0]], r1, s1).wait()
        pltpu.make_async_copy(xb_h.at[i2_v.at[0]], r2, s2).wait()

    def store(j, r1, r2):
        b = base0 + j * K
        pltpu.sync_copy(r1, o1_h.at[pl.ds(b, K)])
        pltpu.sync_copy(r2, o2_h.at[pl.ds(b, K)])

    fire(0, r1a, r2a, sa1, sa2)

    def body(g, carry):
        j = 2 * g
        fire(j + 1, r1b, r2b, sb1, sb2)
        wait(r1a, r2a, sa1, sa2)
        store(j, r1a, r2a)
        fire(j + 2, r1a, r2a, sa1, sa2)
        wait(r1b, r2b, sb1, sb2)
        store(j + 1, r1b, r2b)
        return carry

    lax.fori_loop(0, (_GIT - 1) // 2, body, 0)
    wait(r1a, r2a, sa1, sa2)
    store(_GIT - 1, r1a, r2a)


def _sc_gather(t1, xb, row_r, col_r):
    K = _GK
    f = pl.kernel(
        _gather_body,
        out_type=[
            jax.ShapeDtypeStruct((_E, 2, _F), jnp.int32),
            jax.ShapeDtypeStruct((_E, _F), jnp.int32),
        ],
        mesh=plsc.VectorSubcoreMesh(core_axis_name="c", subcore_axis_name="s"),
        scratch_types=[
            pltpu.VMEM((_GIT, K), jnp.int32),
            pltpu.VMEM((_GIT, K), jnp.int32),
            pltpu.VMEM((K, 2, _F), jnp.int32),
            pltpu.VMEM((K, _F), jnp.int32),
            pltpu.VMEM((K, 2, _F), jnp.int32),
            pltpu.VMEM((K, _F), jnp.int32),
            pltpu.SemaphoreType.DMA,
            pltpu.SemaphoreType.DMA,
            pltpu.SemaphoreType.DMA,
            pltpu.SemaphoreType.DMA,
        ],
    )
    return f(t1, xb, row_r, col_r)


def _edge_body(g1_ref, g2_ref, ea_ref, wec_ref, ew2_ref, eb2_ref, wn1b_ref,
               ean_ref, h1n_ref):
    eaw = jnp.dot(ea_ref[...], wec_ref[...], preferred_element_type=_f32)
    t1a_lo, t1a_hi = _unpack2(g1_ref[:, 0, :])
    t1b_lo, t1b_hi = _unpack2(g1_ref[:, 1, :])
    xb_lo, xb_hi = _unpack2(g2_ref[...])
    h1e_lo = jnp.maximum(t1a_lo + xb_lo + eaw[:, :_F], 0.0)
    h1e_hi = jnp.maximum(t1a_hi + xb_hi + eaw[:, _F:], 0.0)
    ean = (jnp.dot(h1e_lo, ew2_ref[:_F], preferred_element_type=_f32)
           + jnp.dot(h1e_hi, ew2_ref[_F:], preferred_element_type=_f32)
           + eb2_ref[...])
    eanw = jnp.dot(ean, wn1b_ref[...], preferred_element_type=_f32)
    ean_ref[...] = ean
    h1n_ref[0] = jnp.maximum(t1b_lo + eanw[:, :_F], 0.0)
    h1n_ref[1] = jnp.maximum(t1b_hi + eanw[:, _F:], 0.0)


def _edge_dense(g1, g2, ea, wec, ew2, eb2, wn1b):
    nb = _E // _BE
    return pl.pallas_call(
        _edge_body,
        grid=(nb,),
        in_specs=[
            pl.BlockSpec((_BE, 2, _F), lambda i: (i, 0, 0)),
            pl.BlockSpec((_BE, _F), lambda i: (i, 0)),
            pl.BlockSpec((_BE, _FE), lambda i: (i, 0)),
            pl.BlockSpec((_FE, _H), lambda i: (0, 0)),
            pl.BlockSpec((_H, _FE), lambda i: (0, 0)),
            pl.BlockSpec((1, _FE), lambda i: (0, 0)),
            pl.BlockSpec((_FE, _H), lambda i: (0, 0)),
        ],
        out_specs=[
            pl.BlockSpec((_BE, _FE), lambda i: (i, 0)),
            pl.BlockSpec((2, _BE, _F), lambda i: (0, i, 0)),
        ],
        out_shape=[
            jax.ShapeDtypeStruct((_E, _FE), _f32),
            jax.ShapeDtypeStruct((2, _E, _F), _f32),
        ],
    )(g1, g2, ea, wec, ew2, eb2, wn1b)


_SK = 80
_SIT = (_E // _NS) // _SK


def _scatter_body(h1n_h, col_h, zer_h, out_h, idx_v, bufa, bufb, acc_sh, sa, sb):
    K = _SK
    rows_per_tile = _NP // _NS
    per_tile = _E // _NS
    c = lax.axis_index("c")
    s = lax.axis_index("s")

    pltpu.sync_copy(zer_h.at[pl.ds(s * rows_per_tile, rows_per_tile)],
                    acc_sh.at[pl.ds(s * rows_per_tile, rows_per_tile)])
    pltpu.sync_copy(col_h.at[s], idx_v)
    plsc.subcore_barrier()

    def fire(j, buf, sem):
        pltpu.async_copy(h1n_h.at[c, pl.ds(s * per_tile + j * K, K)], buf, sem)

    def wait(buf, sem):
        pltpu.make_async_copy(h1n_h.at[c, pl.ds(0, K)], buf, sem).wait()

    def scat(j, buf):
        pltpu.sync_copy(buf, acc_sh.at[idx_v.at[j]], add=True)

    fire(0, bufa, sa)

    def body(g, carry):
        j = 2 * g
        fire(j + 1, bufb, sb)
        wait(bufa, sa)
        scat(j, bufa)
        fire(j + 2, bufa, sa)
        wait(bufb, sb)
        scat(j + 1, bufb)
        return carry

    lax.fori_loop(0, (_SIT - 1) // 2, body, 0)
    wait(bufa, sa)
    scat(_SIT - 1, bufa)

    plsc.subcore_barrier()
    pltpu.sync_copy(acc_sh.at[pl.ds(s * rows_per_tile, rows_per_tile)],
                    out_h.at[c, pl.ds(s * rows_per_tile, rows_per_tile)])


def _sc_scatter(h1n, col_r, zer):
    K = _SK
    f = pl.kernel(
        _scatter_body,
        out_type=jax.ShapeDtypeStruct((2, _NP, _F), _f32),
        mesh=plsc.VectorSubcoreMesh(core_axis_name="c", subcore_axis_name="s"),
        scratch_types=[
            pltpu.VMEM((_SIT, K), jnp.int32),
            pltpu.VMEM((K, _F), _f32),
            pltpu.VMEM((K, _F), _f32),
            pltpu.VMEM_SHARED((_NP, _F), _f32),
            pltpu.SemaphoreType.DMA,
            pltpu.SemaphoreType.DMA,
        ],
    )
    return f(h1n, col_r, zer)


def _deg_body(col_h, zer_h, ones_h, out_h, idx_v, ones_v, acc_sh):
    K = 80
    rows_per_tile = _NP // _NS
    per_tile = _E // _NS
    c = lax.axis_index("c")
    s = lax.axis_index("s")

    @pl.when(c == 0)
    def _():
        pltpu.sync_copy(ones_h, ones_v)
        pltpu.sync_copy(zer_h.at[pl.ds(s * rows_per_tile, rows_per_tile)],
                        acc_sh.at[pl.ds(s * rows_per_tile, rows_per_tile)])
        pltpu.sync_copy(col_h.at[s], idx_v)
        plsc.subcore_barrier()

        def body(i, carry):
            pltpu.sync_copy(ones_v, acc_sh.at[idx_v.at[i]], add=True)
            return carry

        lax.fori_loop(0, per_tile // K, body, 0)
        plsc.subcore_barrier()
        pltpu.sync_copy(acc_sh.at[pl.ds(s * rows_per_tile, rows_per_tile)],
                        out_h.at[pl.ds(s * rows_per_tile, rows_per_tile)])


def _sc_degree(col_r, zer):
    K = 80
    ones = jnp.ones((K, _F), _f32)
    f = pl.kernel(
        _deg_body,
        out_type=jax.ShapeDtypeStruct((_NP, _F), _f32),
        mesh=plsc.VectorSubcoreMesh(core_axis_name="c", subcore_axis_name="s"),
        scratch_types=[
            pltpu.VMEM((_SIT, K), jnp.int32),
            pltpu.VMEM((K, _F), _f32),
            pltpu.VMEM_SHARED((_NP, _F), _f32),
        ],
    )
    return f(col_r, zer, ones)


def _node_post_body(x_ref, agg_ref, deg_ref, oh_ref, u_ref,
                    n1w2_ref, n1b2_ref, wn2a_ref, wn2b_ref, wn2c_ref,
                    n2b1_ref, n2w2_ref, n2b2_ref, xn_ref, xs_ref):
    deg = deg_ref[:, :1]
    aggf = jnp.concatenate([agg_ref[0], agg_ref[1]], axis=1)
    aggm = aggf / jnp.maximum(deg, 1.0)
    aggv = (jnp.dot(aggm, n1w2_ref[...], preferred_element_type=_f32)
            + n1b2_ref[...] * jnp.minimum(deg, 1.0))
    ub = jnp.dot(oh_ref[...],
                 jnp.dot(u_ref[...], wn2c_ref[...], preferred_element_type=_f32),
                 preferred_element_type=_f32)
    h2 = jnp.maximum(
        jnp.dot(x_ref[...], wn2a_ref[...], preferred_element_type=_f32)
        + jnp.dot(aggv, wn2b_ref[...], preferred_element_type=_f32)
        + ub + n2b1_ref[...], 0.0)
    xn = jnp.dot(h2, n2w2_ref[...], preferred_element_type=_f32) + n2b2_ref[...]
    xn_ref[...] = xn
    part = lax.dot_general(
        oh_ref[...], jnp.concatenate([xn, jnp.ones((_BN, _F), _f32)], axis=1),
        (((0,), (0,)), ((), ())), preferred_element_type=_f32)
    i = pl.program_id(0)

    @pl.when(i == 0)
    def _():
        xs_ref[...] = jnp.zeros((_G, 2 * _F), _f32)

    xs_ref[...] += part


def _node_post(x, agg, deg16, oh, u, n1w2, n1b2, wn2a, wn2b, wn2c, n2b1, n2w2, n2b2):
    nb = _N // _BN
    return pl.pallas_call(
        _node_post_body,
        grid=(nb,),
        in_specs=[
            pl.BlockSpec((_BN, _F), lambda i: (i, 0)),
            pl.BlockSpec((2, _BN, _F), lambda i: (0, i, 0)),
            pl.BlockSpec((_BN, _F), lambda i: (i, 0)),
            pl.BlockSpec((_BN, _G), lambda i: (i, 0)),
            pl.BlockSpec((_G, _FG), lambda i: (0, 0)),
            pl.BlockSpec((_H, _H), lambda i: (0, 0)),
            pl.BlockSpec((1, _H), lambda i: (0, 0)),
            pl.BlockSpec((_F, _H), lambda i: (0, 0)),
            pl.BlockSpec((_H, _H), lambda i: (0, 0)),
            pl.BlockSpec((_FG, _H), lambda i: (0, 0)),
            pl.BlockSpec((1, _H), lambda i: (0, 0)),
            pl.BlockSpec((_H, _F), lambda i: (0, 0)),
            pl.BlockSpec((1, _F), lambda i: (0, 0)),
        ],
        out_specs=[
            pl.BlockSpec((_BN, _F), lambda i: (i, 0)),
            pl.BlockSpec((_G, 2 * _F), lambda i: (0, 0)),
        ],
        out_shape=[
            jax.ShapeDtypeStruct((_N, _F), _f32),
            jax.ShapeDtypeStruct((_G, 2 * _F), _f32),
        ],
    )(x, agg, deg16, oh, u, n1w2, n1b2, wn2a, wn2b, wn2c, n2b1, n2w2, n2b2)


def _glob_mlp(xs, u_ref, wga_ref, wgb_ref, gb1_ref, gw2_ref, gb2_ref):
    cnt = xs[:, _F:_F + 1]
    xm = xs[:, :_F] / jnp.maximum(cnt, 1.0)
    h = jnp.maximum(
        jnp.dot(u_ref[...], wga_ref[...], preferred_element_type=_f32)
        + jnp.dot(xm, wgb_ref[...], preferred_element_type=_f32)
        + gb1_ref[...], 0.0)
    return jnp.dot(h, gw2_ref[...], preferred_element_type=_f32) + gb2_ref[...]


def _glob_body(xs_ref, u_ref, wga_ref, wgb_ref, gb1_ref, gw2_ref,
               gb2_ref, un_ref):
    un_ref[...] = _glob_mlp(xs_ref[...], u_ref, wga_ref, wgb_ref, gb1_ref,
                            gw2_ref, gb2_ref)


def _glob_update(xs2, u, wga, wgb, gb1, gw2, gb2):
    return pl.pallas_call(
        _glob_body,
        out_shape=jax.ShapeDtypeStruct((_G, _FG), _f32),
    )(xs2, u, wga, wgb, gb1, gw2, gb2)


def _pre_glob_body(x_ref, oh_ref, u_ref, xs_ref, wga_ref, wgb_ref, gb1_ref,
                   gw2_ref, gb2_ref, wcat_ref, wed_ref, eb1_ref, n1b1_ref,
                   t1_ref, xb_ref, un_ref):
    un = _glob_mlp(xs_ref[...], u_ref, wga_ref, wgb_ref, gb1_ref, gw2_ref, gb2_ref)
    un_ref[...] = un
    x = x_ref[...]
    xc = jnp.dot(x, wcat_ref[...], preferred_element_type=_f32)
    ug2 = jnp.dot(un, wed_ref[...], preferred_element_type=_f32) + eb1_ref[...]
    t1a = xc[:, :_H] + jnp.dot(oh_ref[...], ug2, preferred_element_type=_f32)
    t1b = xc[:, _H:2 * _H] + n1b1_ref[...]
    t1_ref[:, 0, :] = _pack2(t1a[:, :_F], t1a[:, _F:])
    t1_ref[:, 1, :] = _pack2(t1b[:, :_F], t1b[:, _F:])
    xb_ref[...] = _pack2(xc[:, 2 * _H:2 * _H + _F], xc[:, 2 * _H + _F:])


def _node_pre_glob(x, oh, u, xs2, wga, wgb, gb1, gw2, gb2, wcat, wed, eb1, n1b1):
    nb = _N // _BN
    return pl.pallas_call(
        _pre_glob_body,
        grid=(nb,),
        in_specs=[
            pl.BlockSpec((_BN, _F), lambda i: (i, 0)),
            pl.BlockSpec((_BN, _G), lambda i: (i, 0)),
            pl.BlockSpec((_G, _FG), lambda i: (0, 0)),
            pl.BlockSpec((_G, 2 * _F), lambda i: (0, 0)),
            pl.BlockSpec((_FG, _H), lambda i: (0, 0)),
            pl.BlockSpec((_F, _H), lambda i: (0, 0)),
            pl.BlockSpec((1, _H), lambda i: (0, 0)),
            pl.BlockSpec((_H, _FG), lambda i: (0, 0)),
            pl.BlockSpec((1, _FG), lambda i: (0, 0)),
            pl.BlockSpec((_F, 3 * _H), lambda i: (0, 0)),
            pl.BlockSpec((_FG, _H), lambda i: (0, 0)),
            pl.BlockSpec((1, _H), lambda i: (0, 0)),
            pl.BlockSpec((1, _H), lambda i: (0, 0)),
        ],
        out_specs=[
            pl.BlockSpec((_BN, 2, _F), lambda i: (i, 0, 0)),
            pl.BlockSpec((_BN, _F), lambda i: (i, 0)),
            pl.BlockSpec((_G, _FG), lambda i: (0, 0)),
        ],
        out_shape=[
            jax.ShapeDtypeStruct((_N, 2, _F), jnp.int32),
            jax.ShapeDtypeStruct((_N, _F), jnp.int32),
            jax.ShapeDtypeStruct((_G, _FG), _f32),
        ],
    )(x, oh, u, xs2, wga, wgb, gb1, gw2, gb2, wcat, wed, eb1, n1b1)


def kernel(x, edge_index, edge_attr, u, batch,
           edge_w1, edge_b1, edge_w2, edge_b2,
           node1_w1, node1_b1, node1_w2, node1_b2,
           node2_w1, node2_b1, node2_w2, node2_b2,
           glob_w1, glob_b1, glob_w2, glob_b2):
    row = edge_index[0].astype(jnp.int32)
    col = edge_index[1].astype(jnp.int32)
    row_g = row.reshape(_NW, _GIT, _GK)
    col_g = col.reshape(_NW, _GIT, _GK)
    col_s = col.reshape(_NS, _SIT, _SK)

    oh = (batch[:, None] == jnp.arange(_G, dtype=batch.dtype)[None, :]).astype(_f32)

    wea = edge_w1[:_F]
    web = edge_w1[_F:2 * _F]
    wec = edge_w1[2 * _F:2 * _F + _FE]
    wed = edge_w1[2 * _F + _FE:]
    wn1a = node1_w1[:_F]
    wn1b = node1_w1[_F:]
    wn2a = node2_w1[:_F]
    wn2b = node2_w1[_F:_F + _H]
    wn2c = node2_w1[_F + _H:]
    wga = glob_w1[:_FG]
    wgb = glob_w1[_FG:]
    wcat = jnp.concatenate([wea, wn1a, web], axis=1)

    eb1 = edge_b1[None, :]
    n1b1 = node1_b1[None, :]
    eb2 = edge_b2[None, :]
    n1b2 = node1_b2[None, :]
    n2b1 = node2_b1[None, :]
    n2b2 = node2_b2[None, :]
    gb1 = glob_b1[None, :]
    gb2 = glob_b2[None, :]

    zer = jnp.zeros((_NP, _F), _f32)

    deg16 = _sc_degree(col_s, zer)

    t1, xb = _node_pre(x, oh, u, wcat, wed, eb1, n1b1)
    for s in range(3):
        g1, g2 = _sc_gather(t1, xb, row_g, col_g)
        ean, h1n = _edge_dense(g1, g2, edge_attr, wec, edge_w2, eb2, wn1b)
        agg = _sc_scatter(h1n, col_s, zer)
        x, xs2 = _node_post(x, agg, deg16, oh, u, node1_w2, n1b2,
                            wn2a, wn2b, wn2c, n2b1, node2_w2, n2b2)
        edge_attr = ean
        if s < 2:
            t1, xb, u = _node_pre_glob(x, oh, u, xs2, wga, wgb, gb1,
                                       glob_w2, gb2, wcat, wed, eb1, n1b1)
        else:
            u = _glob_update(xs2, u, wga, wgb, gb1, glob_w2, gb2)

    return (x, edge_attr, u)


# speedup vs baseline: 1.5672x; 1.1519x over previous
import functools

import jax
import jax.numpy as jnp
from jax import lax
from jax.experimental import pallas as pl
from jax.experimental.pallas import tpu as pltpu
from jax.experimental.pallas import tpu_sc as plsc

_N = 10000
_E = 160000
_F = 128
_FE = 16
_FG = 16
_H = 256
_G = 64

_NP = 10240
_BN = 2000
_BE = 2000
_NC = 2
_NS = 16
_NW = _NC * _NS

_f32 = jnp.float32


def _pack2(lo, hi):
    lo16 = lax.bitcast_convert_type(lo.astype(jnp.bfloat16), jnp.uint16).astype(jnp.uint32)
    hi16 = lax.bitcast_convert_type(hi.astype(jnp.bfloat16), jnp.uint16).astype(jnp.uint32)
    return lax.bitcast_convert_type((hi16 << 16) | lo16, jnp.int32)


def _unpack2(v):
    u = lax.bitcast_convert_type(v, jnp.uint32)
    lo = lax.bitcast_convert_type(u << 16, _f32)
    hi = lax.bitcast_convert_type(u & jnp.uint32(0xFFFF0000), _f32)
    return lo, hi


def _node_pre_body(x_ref, oh_ref, u_ref, wcat_ref, wed_ref, eb1_ref,
                   n1b1_ref, t1_ref, xb_ref):
    x = x_ref[...]
    xc = jnp.dot(x, wcat_ref[...], preferred_element_type=_f32)
    ug2 = jnp.dot(u_ref[...], wed_ref[...], preferred_element_type=_f32) + eb1_ref[...]
    t1a = xc[:, :_H] + jnp.dot(oh_ref[...], ug2, preferred_element_type=_f32)
    t1b = xc[:, _H:2 * _H] + n1b1_ref[...]
    t1_ref[:, 0, :] = _pack2(t1a[:, :_F], t1a[:, _F:])
    t1_ref[:, 1, :] = _pack2(t1b[:, :_F], t1b[:, _F:])
    xb_ref[...] = _pack2(xc[:, 2 * _H:2 * _H + _F], xc[:, 2 * _H + _F:])


def _node_pre(x, oh, u, wcat, wed, eb1, n1b1):
    nb = _N // _BN
    return pl.pallas_call(
        _node_pre_body,
        grid=(nb,),
        in_specs=[
            pl.BlockSpec((_BN, _F), lambda i: (i, 0)),
            pl.BlockSpec((_BN, _G), lambda i: (i, 0)),
            pl.BlockSpec((_G, _FG), lambda i: (0, 0)),
            pl.BlockSpec((_F, 3 * _H), lambda i: (0, 0)),
            pl.BlockSpec((_FG, _H), lambda i: (0, 0)),
            pl.BlockSpec((1, _H), lambda i: (0, 0)),
            pl.BlockSpec((1, _H), lambda i: (0, 0)),
        ],
        out_specs=[
            pl.BlockSpec((_BN, 2, _F), lambda i: (i, 0, 0)),
            pl.BlockSpec((_BN, _F), lambda i: (i, 0)),
        ],
        out_shape=[
            jax.ShapeDtypeStruct((_N, 2, _F), jnp.int32),
            jax.ShapeDtypeStruct((_N, _F), jnp.int32),
        ],
    )(x, oh, u, wcat, wed, eb1, n1b1)


_GK = 40
_E1 = 80640
_E2 = _E - _E1


def _gather_body(git, t1_h, xb_h, row_h, col_h, o1_h, o2_h,
                 i1_v, i2_v, r1a, r2a, r1b, r2b, sa1, sa2, sb1, sb2):
    K = _GK
    per_w = git * K
    wid = lax.axis_index("s") * _NC + lax.axis_index("c")
    base0 = wid * per_w

    pltpu.sync_copy(row_h.at[wid], i1_v)
    pltpu.sync_copy(col_h.at[wid], i2_v)

    def fire(j, r1, r2, s1, s2):
        pltpu.async_copy(t1_h.at[i1_v.at[j]], r1, s1)
        pltpu.async_copy(xb_h.at[i2_v.at[j]], r2, s2)

    def wait(r1, r2, s1, s2):
        pltpu.make_async_copy(t1_h.at[i1_v.at[0]], r1, s1).wait()
        pltpu.make_async_copy(xb_h.at[i2_v.at[0]], r2, s2).wait()

    def store(j, r1, r2):
        b = base0 + j * K
        pltpu.sync_copy(r1, o1_h.at[pl.ds(b, K)])
        pltpu.sync_copy(r2, o2_h.at[pl.ds(b, K)])

    fire(0, r1a, r2a, sa1, sa2)

    def body(g, carry):
        j = 2 * g
        fire(j + 1, r1b, r2b, sb1, sb2)
        wait(r1a, r2a, sa1, sa2)
        store(j, r1a, r2a)
        fire(j + 2, r1a, r2a, sa1, sa2)
        wait(r1b, r2b, sb1, sb2)
        store(j + 1, r1b, r2b)
        return carry

    if git % 2 == 1:
        lax.fori_loop(0, (git - 1) // 2, body, 0)
        wait(r1a, r2a, sa1, sa2)
        store(git - 1, r1a, r2a)
    else:
        lax.fori_loop(0, git // 2 - 1, body, 0)
        fire(git - 1, r1b, r2b, sb1, sb2)
        wait(r1a, r2a, sa1, sa2)
        store(git - 2, r1a, r2a)
        wait(r1b, r2b, sb1, sb2)
        store(git - 1, r1b, r2b)


def _sc_gather(t1, xb, row_r, col_r, ne):
    K = _GK
    git = ne // (_NW * K)
    f = pl.kernel(
        functools.partial(_gather_body, git),
        out_type=[
            jax.ShapeDtypeStruct((ne, 2, _F), jnp.int32),
            jax.ShapeDtypeStruct((ne, _F), jnp.int32),
        ],
        mesh=plsc.VectorSubcoreMesh(core_axis_name="c", subcore_axis_name="s"),
        scratch_types=[
            pltpu.VMEM((git, K), jnp.int32),
            pltpu.VMEM((git, K), jnp.int32),
            pltpu.VMEM((K, 2, _F), jnp.int32),
            pltpu.VMEM((K, _F), jnp.int32),
            pltpu.VMEM((K, 2, _F), jnp.int32),
            pltpu.VMEM((K, _F), jnp.int32),
            pltpu.SemaphoreType.DMA,
            pltpu.SemaphoreType.DMA,
            pltpu.SemaphoreType.DMA,
            pltpu.SemaphoreType.DMA,
        ],
    )
    return f(t1, xb, row_r, col_r)


def _edge_body(g1_ref, g2_ref, ea_ref, wec_ref, ew2_ref, eb2_ref, wn1b_ref,
               ean_ref, h1n_ref):
    eaw = jnp.dot(ea_ref[...], wec_ref[...], preferred_element_type=_f32)
    t1a_lo, t1a_hi = _unpack2(g1_ref[:, 0, :])
    t1b_lo, t1b_hi = _unpack2(g1_ref[:, 1, :])
    xb_lo, xb_hi = _unpack2(g2_ref[...])
    h1e_lo = jnp.maximum(t1a_lo + xb_lo + eaw[:, :_F], 0.0)
    h1e_hi = jnp.maximum(t1a_hi + xb_hi + eaw[:, _F:], 0.0)
    ean = (jnp.dot(h1e_lo, ew2_ref[:_F], preferred_element_type=_f32)
           + jnp.dot(h1e_hi, ew2_ref[_F:], preferred_element_type=_f32)
           + eb2_ref[...])
    eanw = jnp.dot(ean, wn1b_ref[...], preferred_element_type=_f32)
    ean_ref[...] = ean
    h1n_ref[0] = jnp.maximum(t1b_lo + eanw[:, :_F], 0.0)
    h1n_ref[1] = jnp.maximum(t1b_hi + eanw[:, _F:], 0.0)


def _edge_dense(g1, g2, ea, wec, ew2, eb2, wn1b, ne, be):
    nb = ne // be
    return pl.pallas_call(
        _edge_body,
        grid=(nb,),
        in_specs=[
            pl.BlockSpec((be, 2, _F), lambda i: (i, 0, 0)),
            pl.BlockSpec((be, _F), lambda i: (i, 0)),
            pl.BlockSpec((be, _FE), lambda i: (i, 0)),
            pl.BlockSpec((_FE, _H), lambda i: (0, 0)),
            pl.BlockSpec((_H, _FE), lambda i: (0, 0)),
            pl.BlockSpec((1, _FE), lambda i: (0, 0)),
            pl.BlockSpec((_FE, _H), lambda i: (0, 0)),
        ],
        out_specs=[
            pl.BlockSpec((be, _FE), lambda i: (i, 0)),
            pl.BlockSpec((2, be, _F), lambda i: (0, i, 0)),
        ],
        out_shape=[
            jax.ShapeDtypeStruct((ne, _FE), _f32),
            jax.ShapeDtypeStruct((2, ne, _F), _f32),
        ],
    )(g1, g2, ea, wec, ew2, eb2, wn1b)


_SK = 80
_SIT = (_E // _NS) // _SK


def _scatter_body(sit, h1n_h, col_h, init_h, out_h, idx_v, bufa, bufb, acc_sh, sa, sb):
    K = _SK
    rows_per_tile = _NP // _NS
    per_tile = sit * K
    c = lax.axis_index("c")
    s = lax.axis_index("s")

    pltpu.sync_copy(init_h.at[c, pl.ds(s * rows_per_tile, rows_per_tile)],
                    acc_sh.at[pl.ds(s * rows_per_tile, rows_per_tile)])
    pltpu.sync_copy(col_h.at[s], idx_v)
    plsc.subcore_barrier()

    def fire(j, buf, sem):
        pltpu.async_copy(h1n_h.at[c, pl.ds(s * per_tile + j * K, K)], buf, sem)

    def wait(buf, sem):
        pltpu.make_async_copy(h1n_h.at[c, pl.ds(0, K)], buf, sem).wait()

    def scat(j, buf):
        pltpu.sync_copy(buf, acc_sh.at[idx_v.at[j]], add=True)

    fire(0, bufa, sa)

    def body(g, carry):
        j = 2 * g
        fire(j + 1, bufb, sb)
        wait(bufa, sa)
        scat(j, bufa)
        fire(j + 2, bufa, sa)
        wait(bufb, sb)
        scat(j + 1, bufb)
        return carry

    if sit % 2 == 1:
        lax.fori_loop(0, (sit - 1) // 2, body, 0)
        wait(bufa, sa)
        scat(sit - 1, bufa)
    else:
        lax.fori_loop(0, sit // 2 - 1, body, 0)
        fire(sit - 1, bufb, sb)
        wait(bufa, sa)
        scat(sit - 2, bufa)
        wait(bufb, sb)
        scat(sit - 1, bufb)

    plsc.subcore_barrier()
    pltpu.sync_copy(acc_sh.at[pl.ds(s * rows_per_tile, rows_per_tile)],
                    out_h.at[c, pl.ds(s * rows_per_tile, rows_per_tile)])


def _sc_scatter(h1n, col_r, init, ne):
    K = _SK
    sit = ne // (_NS * K)
    f = pl.kernel(
        functools.partial(_scatter_body, sit),
        out_type=jax.ShapeDtypeStruct((2, _NP, _F), _f32),
        mesh=plsc.VectorSubcoreMesh(core_axis_name="c", subcore_axis_name="s"),
        scratch_types=[
            pltpu.VMEM((sit, K), jnp.int32),
            pltpu.VMEM((K, _F), _f32),
            pltpu.VMEM((K, _F), _f32),
            pltpu.VMEM_SHARED((_NP, _F), _f32),
            pltpu.SemaphoreType.DMA,
            pltpu.SemaphoreType.DMA,
        ],
    )
    return f(h1n, col_r, init)


def _deg_body(col_h, zer_h, ones_h, out_h, idx_v, ones_v, acc_sh):
    K = 80
    rows_per_tile = _NP // _NS
    per_tile = _E // _NS
    c = lax.axis_index("c")
    s = lax.axis_index("s")

    @pl.when(c == 0)
    def _():
        pltpu.sync_copy(ones_h, ones_v)
        pltpu.sync_copy(zer_h.at[pl.ds(s * rows_per_tile, rows_per_tile)],
                        acc_sh.at[pl.ds(s * rows_per_tile, rows_per_tile)])
        pltpu.sync_copy(col_h.at[s], idx_v)
        plsc.subcore_barrier()

        def body(i, carry):
            pltpu.sync_copy(ones_v, acc_sh.at[idx_v.at[i]], add=True)
            return carry

        lax.fori_loop(0, per_tile // K, body, 0)
        plsc.subcore_barrier()
        pltpu.sync_copy(acc_sh.at[pl.ds(s * rows_per_tile, rows_per_tile)],
                        out_h.at[pl.ds(s * rows_per_tile, rows_per_tile)])


def _sc_degree(col_r, zer):
    K = 80
    ones = jnp.ones((K, _F), _f32)
    f = pl.kernel(
        _deg_body,
        out_type=jax.ShapeDtypeStruct((_NP, _F), _f32),
        mesh=plsc.VectorSubcoreMesh(core_axis_name="c", subcore_axis_name="s"),
        scratch_types=[
            pltpu.VMEM((_SIT, K), jnp.int32),
            pltpu.VMEM((K, _F), _f32),
            pltpu.VMEM_SHARED((_NP, _F), _f32),
        ],
    )
    return f(col_r, zer, ones)


def _node_post_body(x_ref, agg_ref, deg_ref, oh_ref, u_ref,
                    n1w2_ref, n1b2_ref, wn2a_ref, wn2b_ref, wn2c_ref,
                    n2b1_ref, n2w2_ref, n2b2_ref, xn_ref, xs_ref):
    deg = deg_ref[:, :1]
    aggf = jnp.concatenate([agg_ref[0], agg_ref[1]], axis=1)
    aggm = aggf / jnp.maximum(deg, 1.0)
    aggv = (jnp.dot(aggm, n1w2_ref[...], preferred_element_type=_f32)
            + n1b2_ref[...] * jnp.minimum(deg, 1.0))
    ub = jnp.dot(oh_ref[...],
                 jnp.dot(u_ref[...], wn2c_ref[...], preferred_element_type=_f32),
                 preferred_element_type=_f32)
    h2 = jnp.maximum(
        jnp.dot(x_ref[...], wn2a_ref[...], preferred_element_type=_f32)
        + jnp.dot(aggv, wn2b_ref[...], preferred_element_type=_f32)
        + ub + n2b1_ref[...], 0.0)
    xn = jnp.dot(h2, n2w2_ref[...], preferred_element_type=_f32) + n2b2_ref[...]
    xn_ref[...] = xn
    part = lax.dot_general(
        oh_ref[...], jnp.concatenate([xn, jnp.ones((_BN, _F), _f32)], axis=1),
        (((0,), (0,)), ((), ())), preferred_element_type=_f32)
    i = pl.program_id(0)

    @pl.when(i == 0)
    def _():
        xs_ref[...] = jnp.zeros((_G, 2 * _F), _f32)

    xs_ref[...] += part


def _node_post(x, agg, deg16, oh, u, n1w2, n1b2, wn2a, wn2b, wn2c, n2b1, n2w2, n2b2):
    nb = _N // _BN
    return pl.pallas_call(
        _node_post_body,
        grid=(nb,),
        in_specs=[
            pl.BlockSpec((_BN, _F), lambda i: (i, 0)),
            pl.BlockSpec((2, _BN, _F), lambda i: (0, i, 0)),
            pl.BlockSpec((_BN, _F), lambda i: (i, 0)),
            pl.BlockSpec((_BN, _G), lambda i: (i, 0)),
            pl.BlockSpec((_G, _FG), lambda i: (0, 0)),
            pl.BlockSpec((_H, _H), lambda i: (0, 0)),
            pl.BlockSpec((1, _H), lambda i: (0, 0)),
            pl.BlockSpec((_F, _H), lambda i: (0, 0)),
            pl.BlockSpec((_H, _H), lambda i: (0, 0)),
            pl.BlockSpec((_FG, _H), lambda i: (0, 0)),
            pl.BlockSpec((1, _H), lambda i: (0, 0)),
            pl.BlockSpec((_H, _F), lambda i: (0, 0)),
            pl.BlockSpec((1, _F), lambda i: (0, 0)),
        ],
        out_specs=[
            pl.BlockSpec((_BN, _F), lambda i: (i, 0)),
            pl.BlockSpec((_G, 2 * _F), lambda i: (0, 0)),
        ],
        out_shape=[
            jax.ShapeDtypeStruct((_N, _F), _f32),
            jax.ShapeDtypeStruct((_G, 2 * _F), _f32),
        ],
    )(x, agg, deg16, oh, u, n1w2, n1b2, wn2a, wn2b, wn2c, n2b1, n2w2, n2b2)


def _glob_mlp(xs, u_ref, wga_ref, wgb_ref, gb1_ref, gw2_ref, gb2_ref):
    cnt = xs[:, _F:_F + 1]
    xm = xs[:, :_F] / jnp.maximum(cnt, 1.0)
    h = jnp.maximum(
        jnp.dot(u_ref[...], wga_ref[...], preferred_element_type=_f32)
        + jnp.dot(xm, wgb_ref[...], preferred_element_type=_f32)
        + gb1_ref[...], 0.0)
    return jnp.dot(h, gw2_ref[...], preferred_element_type=_f32) + gb2_ref[...]


def _glob_body(xs_ref, u_ref, wga_ref, wgb_ref, gb1_ref, gw2_ref,
               gb2_ref, un_ref):
    un_ref[...] = _glob_mlp(xs_ref[...], u_ref, wga_ref, wgb_ref, gb1_ref,
                            gw2_ref, gb2_ref)


def _glob_update(xs2, u, wga, wgb, gb1, gw2, gb2):
    return pl.pallas_call(
        _glob_body,
        out_shape=jax.ShapeDtypeStruct((_G, _FG), _f32),
    )(xs2, u, wga, wgb, gb1, gw2, gb2)


def _pre_glob_body(x_ref, oh_ref, u_ref, xs_ref, wga_ref, wgb_ref, gb1_ref,
                   gw2_ref, gb2_ref, wcat_ref, wed_ref, eb1_ref, n1b1_ref,
                   t1_ref, xb_ref, un_ref):
    un = _glob_mlp(xs_ref[...], u_ref, wga_ref, wgb_ref, gb1_ref, gw2_ref, gb2_ref)
    un_ref[...] = un
    x = x_ref[...]
    xc = jnp.dot(x, wcat_ref[...], preferred_element_type=_f32)
    ug2 = jnp.dot(un, wed_ref[...], preferred_element_type=_f32) + eb1_ref[...]
    t1a = xc[:, :_H] + jnp.dot(oh_ref[...], ug2, preferred_element_type=_f32)
    t1b = xc[:, _H:2 * _H] + n1b1_ref[...]
    t1_ref[:, 0, :] = _pack2(t1a[:, :_F], t1a[:, _F:])
    t1_ref[:, 1, :] = _pack2(t1b[:, :_F], t1b[:, _F:])
    xb_ref[...] = _pack2(xc[:, 2 * _H:2 * _H + _F], xc[:, 2 * _H + _F:])


def _node_pre_glob(x, oh, u, xs2, wga, wgb, gb1, gw2, gb2, wcat, wed, eb1, n1b1):
    nb = _N // _BN
    return pl.pallas_call(
        _pre_glob_body,
        grid=(nb,),
        in_specs=[
            pl.BlockSpec((_BN, _F), lambda i: (i, 0)),
            pl.BlockSpec((_BN, _G), lambda i: (i, 0)),
            pl.BlockSpec((_G, _FG), lambda i: (0, 0)),
            pl.BlockSpec((_G, 2 * _F), lambda i: (0, 0)),
            pl.BlockSpec((_FG, _H), lambda i: (0, 0)),
            pl.BlockSpec((_F, _H), lambda i: (0, 0)),
            pl.BlockSpec((1, _H), lambda i: (0, 0)),
            pl.BlockSpec((_H, _FG), lambda i: (0, 0)),
            pl.BlockSpec((1, _FG), lambda i: (0, 0)),
            pl.BlockSpec((_F, 3 * _H), lambda i: (0, 0)),
            pl.BlockSpec((_FG, _H), lambda i: (0, 0)),
            pl.BlockSpec((1, _H), lambda i: (0, 0)),
            pl.BlockSpec((1, _H), lambda i: (0, 0)),
        ],
        out_specs=[
            pl.BlockSpec((_BN, 2, _F), lambda i: (i, 0, 0)),
            pl.BlockSpec((_BN, _F), lambda i: (i, 0)),
            pl.BlockSpec((_G, _FG), lambda i: (0, 0)),
        ],
        out_shape=[
            jax.ShapeDtypeStruct((_N, 2, _F), jnp.int32),
            jax.ShapeDtypeStruct((_N, _F), jnp.int32),
            jax.ShapeDtypeStruct((_G, _FG), _f32),
        ],
    )(x, oh, u, xs2, wga, wgb, gb1, gw2, gb2, wcat, wed, eb1, n1b1)


def kernel(x, edge_index, edge_attr, u, batch,
           edge_w1, edge_b1, edge_w2, edge_b2,
           node1_w1, node1_b1, node1_w2, node1_b2,
           node2_w1, node2_b1, node2_w2, node2_b2,
           glob_w1, glob_b1, glob_w2, glob_b2):
    row = edge_index[0].astype(jnp.int32)
    col = edge_index[1].astype(jnp.int32)
    row_g1 = row[:_E1].reshape(_NW, -1, _GK)
    col_g1 = col[:_E1].reshape(_NW, -1, _GK)
    row_g2 = row[_E1:].reshape(_NW, -1, _GK)
    col_g2 = col[_E1:].reshape(_NW, -1, _GK)
    col_s1 = col[:_E1].reshape(_NS, -1, _SK)
    col_s2 = col[_E1:].reshape(_NS, -1, _SK)
    col_sf = col.reshape(_NS, _SIT, _SK)

    oh = (batch[:, None] == jnp.arange(_G, dtype=batch.dtype)[None, :]).astype(_f32)

    wea = edge_w1[:_F]
    web = edge_w1[_F:2 * _F]
    wec = edge_w1[2 * _F:2 * _F + _FE]
    wed = edge_w1[2 * _F + _FE:]
    wn1a = node1_w1[:_F]
    wn1b = node1_w1[_F:]
    wn2a = node2_w1[:_F]
    wn2b = node2_w1[_F:_F + _H]
    wn2c = node2_w1[_F + _H:]
    wga = glob_w1[:_FG]
    wgb = glob_w1[_FG:]
    wcat = jnp.concatenate([wea, wn1a, web], axis=1)

    eb1 = edge_b1[None, :]
    n1b1 = node1_b1[None, :]
    eb2 = edge_b2[None, :]
    n1b2 = node1_b2[None, :]
    n2b1 = node2_b1[None, :]
    n2b2 = node2_b2[None, :]
    gb1 = glob_b1[None, :]
    gb2 = glob_b2[None, :]

    zer = jnp.zeros((_NP, _F), _f32)
    zer2 = jnp.zeros((2, _NP, _F), _f32)

    deg16 = _sc_degree(col_sf, zer)

    ea1 = edge_attr[:_E1]
    ea2 = edge_attr[_E1:]
    t1, xb = _node_pre(x, oh, u, wcat, wed, eb1, n1b1)
    for s in range(3):
        g1a, g2a = _sc_gather(t1, xb, row_g1, col_g1, _E1)
        ean1, h1n1 = _edge_dense(g1a, g2a, ea1, wec, edge_w2, eb2, wn1b,
                                 _E1, 2016)
        g1b, g2b = _sc_gather(t1, xb, row_g2, col_g2, _E2)
        agg1 = _sc_scatter(h1n1, col_s1, zer2, _E1)
        ean2, h1n2 = _edge_dense(g1b, g2b, ea2, wec, edge_w2, eb2, wn1b,
                                 _E2, 1984)
        agg = _sc_scatter(h1n2, col_s2, agg1, _E2)
        x, xs2 = _node_post(x, agg, deg16, oh, u, node1_w2, n1b2,
                            wn2a, wn2b, wn2c, n2b1, node2_w2, n2b2)
        ea1, ea2 = ean1, ean2
        if s < 2:
            t1, xb, u = _node_pre_glob(x, oh, u, xs2, wga, wgb, gb1,
                                       glob_w2, gb2, wcat, wed, eb1, n1b1)
        else:
            u = _glob_update(xs2, u, wga, wgb, gb1, glob_w2, gb2)

    return (x, jnp.concatenate([ea1, ea2], axis=0), u)
